# Initial kernel scaffold; baseline (speedup 1.0000x reference)
import jax
import jax.numpy as jnp
from jax.experimental import pallas as pl


def kernel(np_batch, table):
    raise NotImplementedError("write your pallas kernel here")


# speedup vs baseline: 9.2673x; 9.2673x over previous
import functools

import jax
import jax.numpy as jnp
from jax import lax
from jax.experimental import pallas as pl
from jax.experimental.pallas import tpu as pltpu
from jax.experimental.pallas import tpu_sc as plsc

VOCAB = 100000
EMBED_DIM = 128
BATCH = 4096
SEQ_LEN = 200

B = BATCH * SEQ_LEN
NC, NS = 2, 16
NW = NC * NS
B_PER_W = B // NW
CHUNK = 128
N_CHUNK = B_PER_W // CHUNK

_mesh = plsc.VectorSubcoreMesh(core_axis_name="c", subcore_axis_name="s")


@functools.partial(
    pl.kernel,
    mesh=_mesh,
    out_type=jax.ShapeDtypeStruct((B, EMBED_DIM), jnp.float32),
    scratch_types=[
        pltpu.VMEM((N_CHUNK, CHUNK), jnp.int32),
        pltpu.VMEM((2, CHUNK, EMBED_DIM), jnp.float32),
        pltpu.SemaphoreType.DMA,
        pltpu.SemaphoreType.DMA,
    ],
)
def _gather_kernel(idx_hbm, table_hbm, out_hbm, idx_v, rows_v, gsem0, gsem1):
    wid = lax.axis_index("s") * NC + lax.axis_index("c")
    row0 = wid * N_CHUNK
    base = wid * B_PER_W

    pltpu.sync_copy(idx_hbm.at[pl.ds(row0, N_CHUNK)], idx_v)

    def start(j, buf, sem):
        pltpu.async_copy(table_hbm.at[idx_v.at[j]], rows_v.at[buf], sem)

    def drain(buf, sem):
        pltpu.make_async_copy(table_hbm.at[idx_v.at[0]], rows_v.at[buf], sem).wait()

    def writeback(j, buf):
        pltpu.sync_copy(rows_v.at[buf], out_hbm.at[pl.ds(base + j * CHUNK, CHUNK)])

    start(0, 0, gsem0)

    def body(g, carry):
        j = 2 * g
        start(j + 1, 1, gsem1)
        drain(0, gsem0)
        writeback(j, 0)

        @pl.when(j + 2 < N_CHUNK)
        def _():
            start(j + 2, 0, gsem0)

        drain(1, gsem1)
        writeback(j + 1, 1)
        return carry

    lax.fori_loop(0, N_CHUNK // 2, body, 0)


def kernel(np_batch, table):
    idx = np_batch.astype(jnp.int32).reshape(B // CHUNK, CHUNK)
    out = _gather_kernel(idx, table)
    return out.reshape(BATCH, SEQ_LEN, EMBED_DIM)
